# baseline (speedup 1.0000x reference)
import functools
import math

import jax
import jax.numpy as jnp
from jax import lax
from jax.experimental import pallas as pl
from jax.experimental.pallas import tpu as pltpu
from jax.experimental.pallas import tpu_sc as plsc

_VOCAB = 1_000_000
_D = 64
_BATCH = 4096
_SEQ = 200

_NC = 2
_NS = 16
_NW = _NC * _NS
_BT = _BATCH // _NW
_PSLAB = 8
_NSLAB = _SEQ // _PSLAB
_L = 16


def _make_pe(seq_len, emb_dim):
    position = jnp.arange(seq_len, dtype=jnp.float32)[:, None]
    div_term = jnp.exp(
        jnp.arange(0, emb_dim, 2, dtype=jnp.float32)
        * (-math.log(10000.0) / emb_dim))
    pe = jnp.zeros((seq_len, emb_dim), dtype=jnp.float32)
    pe = pe.at[:, 0::2].set(jnp.sin(position * div_term))
    pe = pe.at[:, 1::2].set(jnp.cos(position * div_term))
    return pe


def _emb_body(xt_hbm, tab2_hbm, peb_hbm, out5_hbm,
              idx_v, idx2_v, buf_a, buf_b, obuf_a, obuf_b, peb_v,
              gsem, wsem):
    bt = lax.axis_index("s") * _NC + lax.axis_index("c")
    b0 = bt * _BT
    iota = lax.iota(jnp.int32, _L)

    def slab_body(sl, carry):
        p0 = sl * _PSLAB
        pltpu.sync_copy(xt_hbm.at[pl.ds(p0, _PSLAB), pl.ds(b0, _BT)], idx_v)
        pltpu.sync_copy(peb_hbm.at[pl.ds(p0, _PSLAB)], peb_v)

        def halve_body(i, c2):
            for g in range(_BT // _L):
                idx2_v[i, pl.ds(g * _L, _L)] = lax.shift_right_logical(
                    idx_v[i, pl.ds(g * _L, _L)], 1)
            return c2

        lax.fori_loop(0, _PSLAB, halve_body, 0)

        bufs = [buf_a, buf_b]
        obufs = [obuf_a, obuf_b]
        gcps = [pltpu.async_copy(tab2_hbm.at[idx2_v.at[0]], buf_a, gsem)]
        wcps = []
        for pi in range(_PSLAB):
            gcps[pi].wait()
            if pi + 1 < _PSLAB:
                gcps.append(pltpu.async_copy(
                    tab2_hbm.at[idx2_v.at[pi + 1]], bufs[(pi + 1) % 2], gsem))
            buf = bufs[pi % 2]
            obuf = obufs[pi % 2]
            if pi >= 2:
                wcps[pi - 2].wait()

            def g_body(g, c3):
                rowv = iota + g * _L
                raw = idx_v[pi, pl.ds(g * _L, _L)]
                colbase = lax.shift_left(
                    lax.bitwise_and(raw, jnp.int32(1)), jnp.int32(6))
                for d in range(_D):
                    vals = plsc.load_gather(
                        buf, [rowv, colbase + jnp.int32(d)])
                    pev = peb_v[pi, pl.ds(d * _L, _L)]
                    obuf[d // 8, d % 8, pl.ds(g * _L, _L)] = vals + pev
                return c3

            lax.fori_loop(0, _BT // _L, g_body, 0)
            wcps.append(pltpu.async_copy(
                obuf, out5_hbm.at[p0 + pi, :, bt], wsem))
        wcps[_PSLAB - 2].wait()
        wcps[_PSLAB - 1].wait()
        return carry

    lax.fori_loop(0, _NSLAB, slab_body, 0)


_emb_call = functools.partial(
    pl.kernel,
    out_type=jax.ShapeDtypeStruct((_SEQ, 8, _NW, 8, 128), jnp.float32),
    mesh=plsc.VectorSubcoreMesh(core_axis_name="c", subcore_axis_name="s"),
    scratch_types=[
        pltpu.VMEM((_PSLAB, _BT), jnp.int32),
        pltpu.VMEM((_PSLAB, _BT), jnp.int32),
        pltpu.VMEM((_BT, 128), jnp.float32),
        pltpu.VMEM((_BT, 128), jnp.float32),
        pltpu.VMEM((8, 8, 128), jnp.float32),
        pltpu.VMEM((8, 8, 128), jnp.float32),
        pltpu.VMEM((_PSLAB, _D * _L), jnp.float32),
        pltpu.SemaphoreType.DMA,
        pltpu.SemaphoreType.DMA,
    ],
    compiler_params=pltpu.CompilerParams(needs_layout_passes=False),
)(_emb_body)


@jax.jit
def kernel(x, table):
    pe = _make_pe(_SEQ, _D)
    peb = jnp.repeat(pe[:, :, None], _L, axis=2).reshape(_SEQ, _D * _L)
    xt = x.T.astype(jnp.int32)
    tab2 = table.reshape(_VOCAB // 2, 128)
    out5 = _emb_call(xt, tab2, peb)
    return out5.transpose(2, 4, 0, 1, 3).reshape(_BATCH, _SEQ, _D)


# speedup vs baseline: 1.3244x; 1.3244x over previous
import functools
import math

import jax
import jax.numpy as jnp
from jax import lax
from jax.experimental import pallas as pl
from jax.experimental.pallas import tpu as pltpu
from jax.experimental.pallas import tpu_sc as plsc

_VOCAB = 1_000_000
_D = 64
_BATCH = 4096
_SEQ = 200

_NC = 2
_NS = 16
_NW = _NC * _NS
_BT = _BATCH // _NW
_PSLAB = 8
_NSLAB = _SEQ // _PSLAB
_L = 16


def _make_pe(seq_len, emb_dim):
    position = jnp.arange(seq_len, dtype=jnp.float32)[:, None]
    div_term = jnp.exp(
        jnp.arange(0, emb_dim, 2, dtype=jnp.float32)
        * (-math.log(10000.0) / emb_dim))
    pe = jnp.zeros((seq_len, emb_dim), dtype=jnp.float32)
    pe = pe.at[:, 0::2].set(jnp.sin(position * div_term))
    pe = pe.at[:, 1::2].set(jnp.cos(position * div_term))
    return pe


def _emb_body(xt_hbm, tab2_hbm, peb_hbm, out5_hbm,
              idx_v, idx2_v, buf_a, buf_b, obuf_a, obuf_b, peb_v,
              gsem, wsem):
    bt = lax.axis_index("s") * _NC + lax.axis_index("c")
    b0 = bt * _BT
    iota = lax.iota(jnp.int32, _L)

    def slab_body(sl, carry):
        p0 = sl * _PSLAB
        pltpu.sync_copy(xt_hbm.at[pl.ds(p0, _PSLAB), pl.ds(b0, _BT)], idx_v)
        pltpu.sync_copy(peb_hbm.at[pl.ds(p0, _PSLAB)], peb_v)

        @plsc.parallel_loop(0, _PSLAB)
        def halve_body(i):
            for g in range(_BT // _L):
                idx2_v[i, pl.ds(g * _L, _L)] = lax.shift_right_logical(
                    idx_v[i, pl.ds(g * _L, _L)], 1)

        def compute(pi, buf):
            @plsc.parallel_loop(0, _BT // _L)
            def g_body(g):
                rowv = iota + g * _L
                raw = idx_v[pi, pl.ds(g * _L, _L)]
                colbase = lax.shift_left(
                    lax.bitwise_and(raw, jnp.int32(1)), jnp.int32(6))
                for d in range(_D):
                    vals = plsc.load_gather(
                        buf, [rowv, colbase + jnp.int32(d)])
                    pev = peb_v[pi, pl.ds(d * _L, _L)]
                    obuf_a[d // 8, d % 8, pl.ds(g * _L, _L)] = vals + pev
            pltpu.sync_copy(obuf_a, out5_hbm.at[p0 + pi, :, bt])

        pltpu.async_copy(tab2_hbm.at[idx2_v.at[0]], buf_a, gsem)

        def pair_body(j, c2):
            pa = 2 * j
            pltpu.make_async_copy(
                tab2_hbm.at[idx2_v.at[pa]], buf_a, gsem).wait()
            pltpu.async_copy(tab2_hbm.at[idx2_v.at[pa + 1]], buf_b, gsem)
            compute(pa, buf_a)
            pltpu.make_async_copy(
                tab2_hbm.at[idx2_v.at[pa + 1]], buf_b, gsem).wait()

            @pl.when(j < _PSLAB // 2 - 1)
            def _():
                pltpu.async_copy(tab2_hbm.at[idx2_v.at[pa + 2]], buf_a, gsem)

            compute(pa + 1, buf_b)
            return c2

        lax.fori_loop(0, _PSLAB // 2, pair_body, 0)
        return carry

    lax.fori_loop(0, _NSLAB, slab_body, 0)


_emb_call = functools.partial(
    pl.kernel,
    out_type=jax.ShapeDtypeStruct((_SEQ, 8, _NW, 8, 128), jnp.float32),
    mesh=plsc.VectorSubcoreMesh(core_axis_name="c", subcore_axis_name="s"),
    scratch_types=[
        pltpu.VMEM((_PSLAB, _BT), jnp.int32),
        pltpu.VMEM((_PSLAB, _BT), jnp.int32),
        pltpu.VMEM((_BT, 128), jnp.float32),
        pltpu.VMEM((_BT, 128), jnp.float32),
        pltpu.VMEM((8, 8, 128), jnp.float32),
        pltpu.VMEM((8, 8, 128), jnp.float32),
        pltpu.VMEM((_PSLAB, _D * _L), jnp.float32),
        pltpu.SemaphoreType.DMA,
        pltpu.SemaphoreType.DMA,
    ],
    compiler_params=pltpu.CompilerParams(needs_layout_passes=False),
)(_emb_body)


@jax.jit
def kernel(x, table):
    pe = _make_pe(_SEQ, _D)
    peb = jnp.repeat(pe[:, :, None], _L, axis=2).reshape(_SEQ, _D * _L)
    xt = x.T.astype(jnp.int32)
    tab2 = table.reshape(_VOCAB // 2, 128)
    out5 = _emb_call(xt, tab2, peb)
    return out5.transpose(2, 4, 0, 1, 3).reshape(_BATCH, _SEQ, _D)
